# Initial kernel scaffold; baseline (speedup 1.0000x reference)
import jax
import jax.numpy as jnp
from jax.experimental import pallas as pl


def kernel(words_idxs, chars_idxs, char_emb, word_emb, conv_w, conv_b, W1, b1, W2, b2):
    raise NotImplementedError("write your pallas kernel here")


# speedup vs baseline: 3.6225x; 3.6225x over previous
import functools

import numpy as np

import jax
import jax.numpy as jnp
from jax.experimental import pallas as pl
from jax.experimental.pallas import tpu as pltpu

_NREP = 512
_NCONV = 384
_NBIG = 512


def _tagger_kernel(cidx_ref, widx_ref, s_ref, t_ref, wbig_ref, w1c_ref,
                   w2_ref, aux_ref, out_ref, *, O, Od):
    idx = jnp.concatenate([cidx_ref[...], widx_ref[...]], axis=1)
    idx16 = idx.astype(jnp.bfloat16)

    rep = jnp.dot(idx16, s_ref[...],
                  preferred_element_type=jnp.float32).astype(jnp.bfloat16)
    oh = jnp.where(rep == t_ref[...], jnp.bfloat16(1.0), jnp.bfloat16(0.0))

    big = jnp.dot(oh, wbig_ref[...], preferred_element_type=jnp.float32)

    g0 = big[:, 0:128]
    g1 = big[:, 128:256]
    g2 = big[:, 256:384]
    gm01 = jnp.maximum(g0, g1)
    gm = jnp.maximum(gm01, g2)
    cf = jnp.maximum(jnp.maximum(gm[:, 0:O], gm[:, O:2 * O]),
                     gm01[:, 2 * O:3 * O]) + aux_ref[0:1, 0:O]

    h = jnp.tanh(big[:, _NCONV:_NBIG]
                 + jnp.dot(cf.astype(jnp.bfloat16), w1c_ref[...],
                           preferred_element_type=jnp.float32)
                 + aux_ref[1:2, :])

    logits = jnp.dot(h.astype(jnp.bfloat16), w2_ref[...],
                     preferred_element_type=jnp.float32) + aux_ref[2:3, :]
    m = jnp.max(logits, axis=-1, keepdims=True)
    e = jnp.exp(logits - m)
    lse = jnp.log(jnp.sum(e, axis=-1, keepdims=True))
    out_ref[...] = (logits - (m + lse))[:, :Od]


@functools.partial(jax.jit, static_argnames=("tile_b", "interpret"))
def _forward(words_idxs, chars_idxs, char_emb, word_emb, conv_w, conv_b,
             W1, b1, W2, b2, *, tile_b=512, interpret=False):
    B, Cs = chars_idxs.shape
    Wn = words_idxs.shape[1]
    char_emb = char_emb.astype(jnp.float32)
    word_emb = word_emb.astype(jnp.float32)
    Vc, L = char_emb.shape
    Vw, E = word_emb.shape
    Wc = conv_w.astype(jnp.float32)
    O = Wc.shape[0]
    W1f = W1.astype(jnp.float32)
    W2f = W2.astype(jnp.float32)
    Od, H = W2f.shape
    Dw = Wn * E
    hi = jax.lax.Precision.HIGHEST

    KI = Cs + Wn
    S_np = np.zeros((KI, _NREP), np.float32)
    T_np = np.full((1, _NREP), -1.0, np.float32)
    for c in range(Cs):
        S_np[c, c * Vc:(c + 1) * Vc] = 1.0
        T_np[0, c * Vc:(c + 1) * Vc] = np.arange(Vc)
    base = Cs * Vc
    for w in range(Wn):
        S_np[Cs + w, base + w * Vw:base + (w + 1) * Vw] = 1.0
        T_np[0, base + w * Vw:base + (w + 1) * Vw] = np.arange(Vw)
    S_c = jnp.asarray(S_np, jnp.bfloat16)
    T_c = jnp.asarray(T_np, jnp.bfloat16)

    Es = jnp.stack([
        jnp.pad(char_emb[:, :L - 1], ((0, 0), (1, 0))),
        char_emb,
        jnp.pad(char_emb[:, 1:], ((0, 0), (0, 1))),
    ], axis=1)
    W3 = jnp.transpose(Wc, (2, 1, 0))
    T4 = jnp.einsum("vkm,kco->cvmo", Es, W3, precision=hi)
    T4 = jnp.pad(T4.reshape(Cs * Vc, L, O), ((0, 0), (0, 1), (0, 0)))
    T4 = jnp.pad(T4.reshape(Cs * Vc, 3, 3 * O),
                 ((0, 0), (0, 0), (0, 128 - 3 * O)))
    rows_char = T4.reshape(Cs * Vc, _NCONV)

    w1w = jnp.einsum("ve,hwe->wvh", word_emb, W1f[:, :Dw].reshape(H, Wn, E),
                     precision=hi).reshape(Wn * Vw, H)

    Wbig = (jnp.zeros((_NREP, _NBIG), jnp.float32)
            .at[0:Cs * Vc, 0:_NCONV].set(rows_char)
            .at[base:base + Wn * Vw, _NCONV:_NCONV + H].set(w1w)
            .astype(jnp.bfloat16))
    W1c = (jnp.zeros((O, 128), jnp.float32)
           .at[:, :H].set(W1f[:, Dw:].T).astype(jnp.bfloat16))
    W2p = (jnp.zeros((128, 128), jnp.float32)
           .at[:H, :Od].set(W2f.T).astype(jnp.bfloat16))
    aux = (jnp.zeros((8, 128), jnp.float32)
           .at[0, :O].set(conv_b.astype(jnp.float32))
           .at[1, :H].set(b1.astype(jnp.float32))
           .at[2, :].set(-1e30)
           .at[2, :Od].set(b2.astype(jnp.float32)))

    TB = min(tile_b, B)
    grid_b = pl.cdiv(B, TB)

    out = pl.pallas_call(
        functools.partial(_tagger_kernel, O=O, Od=Od),
        out_shape=jax.ShapeDtypeStruct((B, Od), jnp.float32),
        grid_spec=pltpu.PrefetchScalarGridSpec(
            num_scalar_prefetch=0,
            grid=(grid_b,),
            in_specs=[
                pl.BlockSpec((TB, Cs), lambda b: (b, 0)),
                pl.BlockSpec((TB, Wn), lambda b: (b, 0)),
                pl.BlockSpec((KI, _NREP), lambda b: (0, 0)),
                pl.BlockSpec((1, _NREP), lambda b: (0, 0)),
                pl.BlockSpec((_NREP, _NBIG), lambda b: (0, 0)),
                pl.BlockSpec((O, 128), lambda b: (0, 0)),
                pl.BlockSpec((128, 128), lambda b: (0, 0)),
                pl.BlockSpec((8, 128), lambda b: (0, 0)),
            ],
            out_specs=pl.BlockSpec((TB, Od), lambda b: (b, 0)),
        ),
        compiler_params=pltpu.CompilerParams(
            dimension_semantics=("parallel",)),
        interpret=interpret,
    )(chars_idxs.astype(jnp.int32), words_idxs.astype(jnp.int32),
      S_c, T_c, Wbig, W1c, W2p, aux)
    return out


def kernel(words_idxs, chars_idxs, char_emb, word_emb, conv_w, conv_b,
           W1, b1, W2, b2):
    return _forward(words_idxs, chars_idxs, char_emb, word_emb,
                    conv_w, conv_b, W1, b1, W2, b2)


# speedup vs baseline: 5.4239x; 1.4973x over previous
import functools

import numpy as np

import jax
import jax.numpy as jnp
from jax.experimental import pallas as pl
from jax.experimental.pallas import tpu as pltpu

_NREP = 512
_NCONV = 384
_NBIG = 512


def _tagger_kernel(cidx_ref, widx_ref, s_ref, t_ref, wbig_ref, w1c_ref,
                   w2_ref, aux_ref, out_ref, *, O, Od):
    idx = jnp.concatenate([cidx_ref[...], widx_ref[...]], axis=1)
    idx16 = idx.astype(jnp.bfloat16)

    rep = jnp.dot(idx16, s_ref[...],
                  preferred_element_type=jnp.float32).astype(jnp.bfloat16)
    oh = jnp.where(rep == t_ref[...], jnp.bfloat16(1.0), jnp.bfloat16(0.0))

    big = jnp.dot(oh, wbig_ref[...], preferred_element_type=jnp.float32)

    g0 = big[:, 0:128]
    g1 = big[:, 128:256]
    g2 = big[:, 256:384]
    gm01 = jnp.maximum(g0, g1)
    gm = jnp.maximum(gm01, g2)
    cf = jnp.maximum(jnp.maximum(gm[:, 0:O], gm[:, O:2 * O]),
                     gm01[:, 2 * O:3 * O]) + aux_ref[0:1, 0:O]

    h = jnp.tanh(big[:, _NCONV:_NBIG]
                 + jnp.dot(cf.astype(jnp.bfloat16), w1c_ref[...],
                           preferred_element_type=jnp.float32)
                 + aux_ref[1:2, :])

    logits = jnp.dot(h.astype(jnp.bfloat16), w2_ref[...],
                     preferred_element_type=jnp.float32) + aux_ref[2:3, :]
    m = jnp.max(logits, axis=-1, keepdims=True)
    e = jnp.exp(logits - m)
    lse = jnp.log(jnp.sum(e, axis=-1, keepdims=True))
    out_ref[...] = (logits - (m + lse))[:, :Od]


@functools.partial(jax.jit, static_argnames=("tile_b", "interpret"))
def _forward(words_idxs, chars_idxs, char_emb, word_emb, conv_w, conv_b,
             W1, b1, W2, b2, *, tile_b=512, interpret=False):
    B, Cs = chars_idxs.shape
    Wn = words_idxs.shape[1]
    char_emb = char_emb.astype(jnp.float32)
    word_emb = word_emb.astype(jnp.float32)
    Vc, L = char_emb.shape
    Vw, E = word_emb.shape
    Wc = conv_w.astype(jnp.float32)
    O = Wc.shape[0]
    W1f = W1.astype(jnp.float32)
    W2f = W2.astype(jnp.float32)
    Od, H = W2f.shape
    Dw = Wn * E
    hi = jax.lax.Precision.HIGHEST

    KI = Cs + Wn
    S_np = np.zeros((KI, _NREP), np.float32)
    T_np = np.full((1, _NREP), -1.0, np.float32)
    for c in range(Cs):
        S_np[c, c * Vc:(c + 1) * Vc] = 1.0
        T_np[0, c * Vc:(c + 1) * Vc] = np.arange(Vc)
    base = Cs * Vc
    for w in range(Wn):
        S_np[Cs + w, base + w * Vw:base + (w + 1) * Vw] = 1.0
        T_np[0, base + w * Vw:base + (w + 1) * Vw] = np.arange(Vw)
    S_c = jnp.asarray(S_np, jnp.bfloat16)
    T_c = jnp.asarray(T_np, jnp.bfloat16)

    Es = jnp.stack([
        jnp.pad(char_emb[:, :L - 1], ((0, 0), (1, 0))),
        char_emb,
        jnp.pad(char_emb[:, 1:], ((0, 0), (0, 1))),
    ], axis=1)
    W3 = jnp.transpose(Wc, (2, 1, 0))
    T4 = jnp.einsum("vkm,kco->cvmo", Es, W3, precision=hi)
    T4 = jnp.pad(T4.reshape(Cs * Vc, L, O), ((0, 0), (0, 1), (0, 0)))
    T4 = jnp.pad(T4.reshape(Cs * Vc, 3, 3 * O),
                 ((0, 0), (0, 0), (0, 128 - 3 * O)))
    rows_char = T4.reshape(Cs * Vc, _NCONV)

    w1w = jnp.einsum("ve,hwe->wvh", word_emb, W1f[:, :Dw].reshape(H, Wn, E),
                     precision=hi).reshape(Wn * Vw, H)

    Wbig = (jnp.zeros((_NREP, _NBIG), jnp.float32)
            .at[0:Cs * Vc, 0:_NCONV].set(rows_char)
            .at[base:base + Wn * Vw, _NCONV:_NCONV + H].set(w1w)
            .astype(jnp.bfloat16))
    W1c = (jnp.zeros((O, 128), jnp.float32)
           .at[:, :H].set(W1f[:, Dw:].T).astype(jnp.bfloat16))
    W2p = (jnp.zeros((128, 128), jnp.float32)
           .at[:H, :Od].set(W2f.T).astype(jnp.bfloat16))
    aux = (jnp.zeros((8, 128), jnp.float32)
           .at[0, :O].set(conv_b.astype(jnp.float32))
           .at[1, :H].set(b1.astype(jnp.float32))
           .at[2, :].set(-1e30)
           .at[2, :Od].set(b2.astype(jnp.float32)))

    TB = min(tile_b, B)
    grid_b = pl.cdiv(B, TB)

    out = pl.pallas_call(
        functools.partial(_tagger_kernel, O=O, Od=Od),
        out_shape=jax.ShapeDtypeStruct((B, Od), jnp.float32),
        grid_spec=pltpu.PrefetchScalarGridSpec(
            num_scalar_prefetch=0,
            grid=(grid_b,),
            in_specs=[
                pl.BlockSpec((TB, Cs), lambda b: (b, 0)),
                pl.BlockSpec((TB, Wn), lambda b: (b, 0)),
                pl.BlockSpec((KI, _NREP), lambda b: (0, 0)),
                pl.BlockSpec((1, _NREP), lambda b: (0, 0)),
                pl.BlockSpec((_NREP, _NBIG), lambda b: (0, 0)),
                pl.BlockSpec((O, 128), lambda b: (0, 0)),
                pl.BlockSpec((128, 128), lambda b: (0, 0)),
                pl.BlockSpec((8, 128), lambda b: (0, 0)),
            ],
            out_specs=pl.BlockSpec((TB, Od), lambda b: (b, 0)),
        ),
        compiler_params=pltpu.CompilerParams(
            dimension_semantics=("parallel",)),
        interpret=interpret,
    )(chars_idxs.astype(jnp.int32), words_idxs.astype(jnp.int32),
      S_c, T_c, Wbig, W1c, W2p, aux)
    return out


def kernel(words_idxs, chars_idxs, char_emb, word_emb, conv_w, conv_b,
           W1, b1, W2, b2):
    return _forward(words_idxs, chars_idxs, char_emb, word_emb,
                    conv_w, conv_b, W1, b1, W2, b2, tile_b=2048)


# speedup vs baseline: 5.6595x; 1.0434x over previous
import functools

import numpy as np

import jax
import jax.numpy as jnp
from jax.experimental import pallas as pl
from jax.experimental.pallas import tpu as pltpu

_NREP = 512
_NCONV = 384
_NBIG = 512


def _tagger_kernel(cidx_ref, widx_ref, s_ref, t_ref, wbig_ref, w1c_ref,
                   w2_ref, aux_ref, out_ref, *, O, Od):
    idx = jnp.concatenate([cidx_ref[...], widx_ref[...]], axis=1)
    idx16 = idx.astype(jnp.bfloat16)

    rep = jnp.dot(idx16, s_ref[...],
                  preferred_element_type=jnp.float32).astype(jnp.bfloat16)
    oh = jnp.where(rep == t_ref[...], jnp.bfloat16(1.0), jnp.bfloat16(0.0))

    big = jnp.dot(oh, wbig_ref[...], preferred_element_type=jnp.float32)

    g0 = big[:, 0:128]
    g1 = big[:, 128:256]
    g2 = big[:, 256:384]
    gm01 = jnp.maximum(g0, g1)
    gm = jnp.maximum(gm01, g2)
    cf = jnp.maximum(jnp.maximum(gm[:, 0:O], gm[:, O:2 * O]),
                     gm01[:, 2 * O:3 * O]) + aux_ref[0:1, 0:O]

    h = jnp.tanh(big[:, _NCONV:_NBIG]
                 + jnp.dot(cf.astype(jnp.bfloat16), w1c_ref[...],
                           preferred_element_type=jnp.float32)
                 + aux_ref[1:2, :])

    logits = jnp.dot(h.astype(jnp.bfloat16), w2_ref[...],
                     preferred_element_type=jnp.float32) + aux_ref[2:3, :]
    m = jnp.max(logits, axis=-1, keepdims=True)
    e = jnp.exp(logits - m)
    lse = jnp.log(jnp.sum(e, axis=-1, keepdims=True))
    out_ref[...] = (logits - (m + lse))[:, :Od]


@functools.partial(jax.jit, static_argnames=("tile_b", "interpret"))
def _forward(words_idxs, chars_idxs, char_emb, word_emb, conv_w, conv_b,
             W1, b1, W2, b2, *, tile_b=512, interpret=False):
    B, Cs = chars_idxs.shape
    Wn = words_idxs.shape[1]
    char_emb = char_emb.astype(jnp.float32)
    word_emb = word_emb.astype(jnp.float32)
    Vc, L = char_emb.shape
    Vw, E = word_emb.shape
    Wc = conv_w.astype(jnp.float32)
    O = Wc.shape[0]
    W1f = W1.astype(jnp.float32)
    W2f = W2.astype(jnp.float32)
    Od, H = W2f.shape
    Dw = Wn * E
    hi = jax.lax.Precision.HIGHEST

    KI = Cs + Wn
    S_np = np.zeros((KI, _NREP), np.float32)
    T_np = np.full((1, _NREP), -1.0, np.float32)
    for c in range(Cs):
        S_np[c, c * Vc:(c + 1) * Vc] = 1.0
        T_np[0, c * Vc:(c + 1) * Vc] = np.arange(Vc)
    base = Cs * Vc
    for w in range(Wn):
        S_np[Cs + w, base + w * Vw:base + (w + 1) * Vw] = 1.0
        T_np[0, base + w * Vw:base + (w + 1) * Vw] = np.arange(Vw)
    S_c = jnp.asarray(S_np, jnp.bfloat16)
    T_c = jnp.asarray(T_np, jnp.bfloat16)

    Es = jnp.stack([
        jnp.pad(char_emb[:, :L - 1], ((0, 0), (1, 0))),
        char_emb,
        jnp.pad(char_emb[:, 1:], ((0, 0), (0, 1))),
    ], axis=1)
    W3 = jnp.transpose(Wc, (2, 1, 0))
    T4 = jnp.einsum("vkm,kco->cvmo", Es, W3, precision=hi)
    T4 = jnp.pad(T4.reshape(Cs * Vc, L, O), ((0, 0), (0, 1), (0, 0)))
    T4 = jnp.pad(T4.reshape(Cs * Vc, 3, 3 * O),
                 ((0, 0), (0, 0), (0, 128 - 3 * O)))
    rows_char = T4.reshape(Cs * Vc, _NCONV)

    w1w = jnp.einsum("ve,hwe->wvh", word_emb, W1f[:, :Dw].reshape(H, Wn, E),
                     precision=hi).reshape(Wn * Vw, H)

    Wbig = (jnp.zeros((_NREP, _NBIG), jnp.float32)
            .at[0:Cs * Vc, 0:_NCONV].set(rows_char)
            .at[base:base + Wn * Vw, _NCONV:_NCONV + H].set(w1w)
            .astype(jnp.bfloat16))
    W1c = (jnp.zeros((O, 128), jnp.float32)
           .at[:, :H].set(W1f[:, Dw:].T).astype(jnp.bfloat16))
    W2p = (jnp.zeros((128, 128), jnp.float32)
           .at[:H, :Od].set(W2f.T).astype(jnp.bfloat16))
    aux = (jnp.zeros((8, 128), jnp.float32)
           .at[0, :O].set(conv_b.astype(jnp.float32))
           .at[1, :H].set(b1.astype(jnp.float32))
           .at[2, :].set(-1e30)
           .at[2, :Od].set(b2.astype(jnp.float32)))

    TB = min(tile_b, B)
    grid_b = pl.cdiv(B, TB)

    out = pl.pallas_call(
        functools.partial(_tagger_kernel, O=O, Od=Od),
        out_shape=jax.ShapeDtypeStruct((B, Od), jnp.float32),
        grid_spec=pltpu.PrefetchScalarGridSpec(
            num_scalar_prefetch=0,
            grid=(grid_b,),
            in_specs=[
                pl.BlockSpec((TB, Cs), lambda b: (b, 0)),
                pl.BlockSpec((TB, Wn), lambda b: (b, 0)),
                pl.BlockSpec((KI, _NREP), lambda b: (0, 0)),
                pl.BlockSpec((1, _NREP), lambda b: (0, 0)),
                pl.BlockSpec((_NREP, _NBIG), lambda b: (0, 0)),
                pl.BlockSpec((O, 128), lambda b: (0, 0)),
                pl.BlockSpec((128, 128), lambda b: (0, 0)),
                pl.BlockSpec((8, 128), lambda b: (0, 0)),
            ],
            out_specs=pl.BlockSpec((TB, Od), lambda b: (b, 0)),
        ),
        compiler_params=pltpu.CompilerParams(
            dimension_semantics=("parallel",)),
        interpret=interpret,
    )(chars_idxs.astype(jnp.int32), words_idxs.astype(jnp.int32),
      S_c, T_c, Wbig, W1c, W2p, aux)
    return out


def kernel(words_idxs, chars_idxs, char_emb, word_emb, conv_w, conv_b,
           W1, b1, W2, b2):
    return _forward(words_idxs, chars_idxs, char_emb, word_emb,
                    conv_w, conv_b, W1, b1, W2, b2, tile_b=4096)


# speedup vs baseline: 6.5027x; 1.1490x over previous
import functools

import numpy as np

import jax
import jax.numpy as jnp
from jax.experimental import pallas as pl
from jax.experimental.pallas import tpu as pltpu

_NREP = 512
_NCONV = 384
_NBIG = 512


def _tagger_kernel(cidx_ref, widx_ref, s_ref, t_ref, wchar_ref, wword_ref,
                   w1c_ref, w2_ref, aux_ref, out_ref, *, O, Od):
    idx = jnp.concatenate([cidx_ref[...], widx_ref[...]], axis=1)
    idx16 = idx.astype(jnp.bfloat16)

    rep = jnp.dot(idx16, s_ref[...],
                  preferred_element_type=jnp.float32).astype(jnp.bfloat16)
    oh = jnp.where(rep == t_ref[...], jnp.bfloat16(1.0), jnp.bfloat16(0.0))

    conv = jnp.dot(oh[:, 0:256], wchar_ref[...],
                   preferred_element_type=jnp.float32)
    wordt = jnp.dot(oh[:, 256:512], wword_ref[...],
                    preferred_element_type=jnp.float32)

    g0 = conv[:, 0:128]
    g1 = conv[:, 128:256]
    g2 = conv[:, 256:384]
    gm01 = jnp.maximum(g0, g1)
    gm = jnp.maximum(gm01, g2)
    cf = jnp.maximum(jnp.maximum(gm[:, 0:O], gm[:, O:2 * O]),
                     gm01[:, 2 * O:3 * O]) + aux_ref[0:1, 0:O]

    h = jnp.tanh(wordt
                 + jnp.dot(cf.astype(jnp.bfloat16), w1c_ref[...],
                           preferred_element_type=jnp.float32)
                 + aux_ref[1:2, :])

    logits = jnp.dot(h.astype(jnp.bfloat16), w2_ref[...],
                     preferred_element_type=jnp.float32) + aux_ref[2:3, :]
    lse = jnp.log(jnp.sum(jnp.exp(logits), axis=-1, keepdims=True))
    out_ref[...] = (logits - lse)[:, :Od]


@functools.partial(jax.jit, static_argnames=("tile_b", "interpret"))
def _forward(words_idxs, chars_idxs, char_emb, word_emb, conv_w, conv_b,
             W1, b1, W2, b2, *, tile_b=512, interpret=False):
    B, Cs = chars_idxs.shape
    Wn = words_idxs.shape[1]
    char_emb = char_emb.astype(jnp.float32)
    word_emb = word_emb.astype(jnp.float32)
    Vc, L = char_emb.shape
    Vw, E = word_emb.shape
    Wc = conv_w.astype(jnp.float32)
    O = Wc.shape[0]
    W1f = W1.astype(jnp.float32)
    W2f = W2.astype(jnp.float32)
    Od, H = W2f.shape
    Dw = Wn * E
    hi = jax.lax.Precision.HIGHEST

    KI = Cs + Wn
    base = 256
    S_np = np.zeros((KI, _NREP), np.float32)
    T_np = np.full((1, _NREP), -1.0, np.float32)
    for c in range(Cs):
        S_np[c, c * Vc:(c + 1) * Vc] = 1.0
        T_np[0, c * Vc:(c + 1) * Vc] = np.arange(Vc)
    for w in range(Wn):
        S_np[Cs + w, base + w * Vw:base + (w + 1) * Vw] = 1.0
        T_np[0, base + w * Vw:base + (w + 1) * Vw] = np.arange(Vw)
    S_c = jnp.asarray(S_np, jnp.bfloat16)
    T_c = jnp.asarray(T_np, jnp.bfloat16)

    Es = jnp.stack([
        jnp.pad(char_emb[:, :L - 1], ((0, 0), (1, 0))),
        char_emb,
        jnp.pad(char_emb[:, 1:], ((0, 0), (0, 1))),
    ], axis=1)
    W3 = jnp.transpose(Wc, (2, 1, 0))
    T4 = jnp.einsum("vkm,kco->cvmo", Es, W3, precision=hi)
    T4 = jnp.pad(T4.reshape(Cs * Vc, L, O), ((0, 0), (0, 1), (0, 0)))
    T4 = jnp.pad(T4.reshape(Cs * Vc, 3, 3 * O),
                 ((0, 0), (0, 0), (0, 128 - 3 * O)))
    rows_char = T4.reshape(Cs * Vc, _NCONV)

    w1w = jnp.einsum("ve,hwe->wvh", word_emb, W1f[:, :Dw].reshape(H, Wn, E),
                     precision=hi).reshape(Wn * Vw, H)

    Wchar = (jnp.zeros((256, _NCONV), jnp.float32)
             .at[0:Cs * Vc, :].set(rows_char).astype(jnp.bfloat16))
    Wword = (jnp.zeros((256, 128), jnp.float32)
             .at[0:Wn * Vw, :H].set(w1w).astype(jnp.bfloat16))
    W1c = (jnp.zeros((O, 128), jnp.float32)
           .at[:, :H].set(W1f[:, Dw:].T).astype(jnp.bfloat16))
    W2p = (jnp.zeros((128, 128), jnp.float32)
           .at[:H, :Od].set(W2f.T).astype(jnp.bfloat16))
    aux = (jnp.zeros((8, 128), jnp.float32)
           .at[0, :O].set(conv_b.astype(jnp.float32))
           .at[1, :H].set(b1.astype(jnp.float32))
           .at[2, :].set(-1e30)
           .at[2, :Od].set(b2.astype(jnp.float32)))

    TB = min(tile_b, B)
    grid_b = pl.cdiv(B, TB)

    out = pl.pallas_call(
        functools.partial(_tagger_kernel, O=O, Od=Od),
        out_shape=jax.ShapeDtypeStruct((B, Od), jnp.float32),
        grid_spec=pltpu.PrefetchScalarGridSpec(
            num_scalar_prefetch=0,
            grid=(grid_b,),
            in_specs=[
                pl.BlockSpec((TB, Cs), lambda b: (b, 0)),
                pl.BlockSpec((TB, Wn), lambda b: (b, 0)),
                pl.BlockSpec((KI, _NREP), lambda b: (0, 0)),
                pl.BlockSpec((1, _NREP), lambda b: (0, 0)),
                pl.BlockSpec((256, _NCONV), lambda b: (0, 0)),
                pl.BlockSpec((256, 128), lambda b: (0, 0)),
                pl.BlockSpec((O, 128), lambda b: (0, 0)),
                pl.BlockSpec((128, 128), lambda b: (0, 0)),
                pl.BlockSpec((8, 128), lambda b: (0, 0)),
            ],
            out_specs=pl.BlockSpec((TB, Od), lambda b: (b, 0)),
        ),
        compiler_params=pltpu.CompilerParams(
            dimension_semantics=("parallel",)),
        interpret=interpret,
    )(chars_idxs.astype(jnp.int32), words_idxs.astype(jnp.int32),
      S_c, T_c, Wchar, Wword, W1c, W2p, aux)
    return out


def kernel(words_idxs, chars_idxs, char_emb, word_emb, conv_w, conv_b,
           W1, b1, W2, b2):
    return _forward(words_idxs, chars_idxs, char_emb, word_emb,
                    conv_w, conv_b, W1, b1, W2, b2, tile_b=4096)


# speedup vs baseline: 6.5724x; 1.0107x over previous
import functools

import numpy as np

import jax
import jax.numpy as jnp
from jax.experimental import pallas as pl
from jax.experimental.pallas import tpu as pltpu

_NREP = 512
_NCONV = 384
_NBIG = 512


def _tagger_kernel(cidx_ref, widx_ref, sc_ref, tc_ref, sw_ref, tw_ref,
                   wchar_ref, wword_ref, w1c_ref, w2_ref, aux_ref, out_ref,
                   *, O, Od):
    rep_c = jnp.dot(cidx_ref[...].astype(jnp.bfloat16), sc_ref[...],
                    preferred_element_type=jnp.float32).astype(jnp.bfloat16)
    oh_c = jnp.where(rep_c == tc_ref[...], jnp.bfloat16(1.0), jnp.bfloat16(0.0))
    rep_w = jnp.dot(widx_ref[...].astype(jnp.bfloat16), sw_ref[...],
                    preferred_element_type=jnp.float32).astype(jnp.bfloat16)
    oh_w = jnp.where(rep_w == tw_ref[...], jnp.bfloat16(1.0), jnp.bfloat16(0.0))

    conv = jnp.dot(oh_c, wchar_ref[...],
                   preferred_element_type=jnp.float32).astype(jnp.bfloat16)
    wordt = jnp.dot(oh_w, wword_ref[...],
                    preferred_element_type=jnp.float32)

    g0 = conv[:, 0:128]
    g1 = conv[:, 128:256]
    g2 = conv[:, 256:384]
    gm01 = jnp.maximum(g0, g1)
    gm = jnp.maximum(gm01, g2)
    cf = jnp.maximum(jnp.maximum(gm[:, 0:O], gm[:, O:2 * O]),
                     gm01[:, 2 * O:3 * O]) + aux_ref[0:1, 0:O].astype(jnp.bfloat16)

    h = jnp.tanh(wordt
                 + jnp.dot(cf, w1c_ref[...],
                           preferred_element_type=jnp.float32)
                 + aux_ref[1:2, :])

    logits = jnp.dot(h.astype(jnp.bfloat16), w2_ref[...],
                     preferred_element_type=jnp.float32) + aux_ref[2:3, :]
    lse = jnp.log(jnp.sum(jnp.exp(logits), axis=-1, keepdims=True))
    out_ref[...] = (logits - lse)[:, :Od]


@functools.partial(jax.jit, static_argnames=("tile_b", "interpret"))
def _forward(words_idxs, chars_idxs, char_emb, word_emb, conv_w, conv_b,
             W1, b1, W2, b2, *, tile_b=512, interpret=False):
    B, Cs = chars_idxs.shape
    Wn = words_idxs.shape[1]
    char_emb = char_emb.astype(jnp.float32)
    word_emb = word_emb.astype(jnp.float32)
    Vc, L = char_emb.shape
    Vw, E = word_emb.shape
    Wc = conv_w.astype(jnp.float32)
    O = Wc.shape[0]
    W1f = W1.astype(jnp.float32)
    W2f = W2.astype(jnp.float32)
    Od, H = W2f.shape
    Dw = Wn * E
    hi = jax.lax.Precision.HIGHEST

    Sc_np = np.zeros((Cs, 256), np.float32)
    Tc_np = np.full((1, 256), -1.0, np.float32)
    for c in range(Cs):
        Sc_np[c, c * Vc:(c + 1) * Vc] = 1.0
        Tc_np[0, c * Vc:(c + 1) * Vc] = np.arange(Vc)
    Sw_np = np.zeros((Wn, 256), np.float32)
    Tw_np = np.full((1, 256), -1.0, np.float32)
    for w in range(Wn):
        Sw_np[w, w * Vw:(w + 1) * Vw] = 1.0
        Tw_np[0, w * Vw:(w + 1) * Vw] = np.arange(Vw)
    S_c = jnp.asarray(Sc_np, jnp.bfloat16)
    T_c = jnp.asarray(Tc_np, jnp.bfloat16)
    S_w = jnp.asarray(Sw_np, jnp.bfloat16)
    T_w = jnp.asarray(Tw_np, jnp.bfloat16)

    Es = jnp.stack([
        jnp.pad(char_emb[:, :L - 1], ((0, 0), (1, 0))),
        char_emb,
        jnp.pad(char_emb[:, 1:], ((0, 0), (0, 1))),
    ], axis=1)
    W3 = jnp.transpose(Wc, (2, 1, 0))
    T4 = jnp.einsum("vkm,kco->cvmo", Es, W3, precision=hi)
    T4 = jnp.pad(T4.reshape(Cs * Vc, L, O), ((0, 0), (0, 1), (0, 0)))
    T4 = jnp.pad(T4.reshape(Cs * Vc, 3, 3 * O),
                 ((0, 0), (0, 0), (0, 128 - 3 * O)))
    rows_char = T4.reshape(Cs * Vc, _NCONV)

    w1w = jnp.einsum("ve,hwe->wvh", word_emb, W1f[:, :Dw].reshape(H, Wn, E),
                     precision=hi).reshape(Wn * Vw, H)

    Wchar = (jnp.zeros((256, _NCONV), jnp.float32)
             .at[0:Cs * Vc, :].set(rows_char).astype(jnp.bfloat16))
    Wword = (jnp.zeros((256, 128), jnp.float32)
             .at[0:Wn * Vw, :H].set(w1w).astype(jnp.bfloat16))
    W1c = (jnp.zeros((O, 128), jnp.float32)
           .at[:, :H].set(W1f[:, Dw:].T).astype(jnp.bfloat16))
    W2p = (jnp.zeros((128, 128), jnp.float32)
           .at[:H, :Od].set(W2f.T).astype(jnp.bfloat16))
    aux = (jnp.zeros((8, 128), jnp.float32)
           .at[0, :O].set(conv_b.astype(jnp.float32))
           .at[1, :H].set(b1.astype(jnp.float32))
           .at[2, :].set(-1e30)
           .at[2, :Od].set(b2.astype(jnp.float32)))

    TB = min(tile_b, B)
    grid_b = pl.cdiv(B, TB)

    out = pl.pallas_call(
        functools.partial(_tagger_kernel, O=O, Od=Od),
        out_shape=jax.ShapeDtypeStruct((B, Od), jnp.float32),
        grid_spec=pltpu.PrefetchScalarGridSpec(
            num_scalar_prefetch=0,
            grid=(grid_b,),
            in_specs=[
                pl.BlockSpec((TB, Cs), lambda b: (b, 0)),
                pl.BlockSpec((TB, Wn), lambda b: (b, 0)),
                pl.BlockSpec((Cs, 256), lambda b: (0, 0)),
                pl.BlockSpec((1, 256), lambda b: (0, 0)),
                pl.BlockSpec((Wn, 256), lambda b: (0, 0)),
                pl.BlockSpec((1, 256), lambda b: (0, 0)),
                pl.BlockSpec((256, _NCONV), lambda b: (0, 0)),
                pl.BlockSpec((256, 128), lambda b: (0, 0)),
                pl.BlockSpec((O, 128), lambda b: (0, 0)),
                pl.BlockSpec((128, 128), lambda b: (0, 0)),
                pl.BlockSpec((8, 128), lambda b: (0, 0)),
            ],
            out_specs=pl.BlockSpec((TB, Od), lambda b: (b, 0)),
        ),
        compiler_params=pltpu.CompilerParams(
            dimension_semantics=("parallel",)),
        interpret=interpret,
    )(chars_idxs.astype(jnp.int32), words_idxs.astype(jnp.int32),
      S_c, T_c, S_w, T_w, Wchar, Wword, W1c, W2p, aux)
    return out


def kernel(words_idxs, chars_idxs, char_emb, word_emb, conv_w, conv_b,
           W1, b1, W2, b2):
    return _forward(words_idxs, chars_idxs, char_emb, word_emb,
                    conv_w, conv_b, W1, b1, W2, b2, tile_b=4096)


# speedup vs baseline: 6.7046x; 1.0201x over previous
import functools

import numpy as np

import jax
import jax.numpy as jnp
from jax.experimental import pallas as pl
from jax.experimental.pallas import tpu as pltpu

_NREP = 512
_NCONV = 384
_NBIG = 512


def _tagger_kernel(cidx_ref, widx_ref, sc_ref, tc_ref, sw_ref, tw_ref,
                   wchar_ref, wword_ref, w1c_ref, w2_ref, aux_ref, out_ref,
                   *, O, Od):
    rep_c = jnp.dot(cidx_ref[...].astype(jnp.bfloat16), sc_ref[...],
                    preferred_element_type=jnp.float32).astype(jnp.bfloat16)
    oh_c = jnp.where(rep_c == tc_ref[...], jnp.bfloat16(1.0), jnp.bfloat16(0.0))
    rep_w = jnp.dot(widx_ref[...].astype(jnp.bfloat16), sw_ref[...],
                    preferred_element_type=jnp.float32).astype(jnp.bfloat16)
    oh_w = jnp.where(rep_w == tw_ref[...], jnp.bfloat16(1.0), jnp.bfloat16(0.0))

    conv = jnp.dot(oh_c, wchar_ref[...],
                   preferred_element_type=jnp.float32).astype(jnp.bfloat16)
    wordt = jnp.dot(oh_w, wword_ref[...],
                    preferred_element_type=jnp.float32)

    g0 = conv[:, 0:128]
    g1 = conv[:, 128:256]
    g2 = conv[:, 256:384]
    gm01 = jnp.maximum(g0, g1)
    gm = jnp.maximum(gm01, g2)
    cf = jnp.maximum(jnp.maximum(gm[:, 0:O], gm[:, O:2 * O]),
                     gm01[:, 2 * O:3 * O]) + aux_ref[0:1, 0:O].astype(jnp.bfloat16)

    h = jnp.tanh(wordt
                 + jnp.dot(cf, w1c_ref[...],
                           preferred_element_type=jnp.float32)
                 + aux_ref[1:2, :])

    logits = jnp.dot(h.astype(jnp.bfloat16), w2_ref[...],
                     preferred_element_type=jnp.float32) + aux_ref[2:3, :]
    lse = jnp.log(jnp.sum(jnp.exp(logits), axis=-1, keepdims=True))
    out_ref[...] = (logits - lse)[:, :Od]


@functools.partial(jax.jit, static_argnames=("tile_b", "interpret"))
def _forward(words_idxs, chars_idxs, char_emb, word_emb, conv_w, conv_b,
             W1, b1, W2, b2, *, tile_b=512, interpret=False):
    B, Cs = chars_idxs.shape
    Wn = words_idxs.shape[1]
    char_emb = char_emb.astype(jnp.float32)
    word_emb = word_emb.astype(jnp.float32)
    Vc, L = char_emb.shape
    Vw, E = word_emb.shape
    Wc = conv_w.astype(jnp.float32)
    O = Wc.shape[0]
    W1f = W1.astype(jnp.float32)
    W2f = W2.astype(jnp.float32)
    Od, H = W2f.shape
    Dw = Wn * E
    hi = jax.lax.Precision.HIGHEST

    Sc_np = np.zeros((Cs, 256), np.float32)
    Tc_np = np.full((1, 256), -1.0, np.float32)
    for c in range(Cs):
        Sc_np[c, c * Vc:(c + 1) * Vc] = 1.0
        Tc_np[0, c * Vc:(c + 1) * Vc] = np.arange(Vc)
    Sw_np = np.zeros((Wn, 256), np.float32)
    Tw_np = np.full((1, 256), -1.0, np.float32)
    for w in range(Wn):
        Sw_np[w, w * Vw:(w + 1) * Vw] = 1.0
        Tw_np[0, w * Vw:(w + 1) * Vw] = np.arange(Vw)
    S_c = jnp.asarray(Sc_np, jnp.bfloat16)
    T_c = jnp.asarray(Tc_np, jnp.bfloat16)
    S_w = jnp.asarray(Sw_np, jnp.bfloat16)
    T_w = jnp.asarray(Tw_np, jnp.bfloat16)

    Es = jnp.stack([
        jnp.pad(char_emb[:, :L - 1], ((0, 0), (1, 0))),
        char_emb,
        jnp.pad(char_emb[:, 1:], ((0, 0), (0, 1))),
    ], axis=1)
    W3 = jnp.transpose(Wc, (2, 1, 0))
    T4 = jnp.einsum("vkm,kco->cvmo", Es, W3, precision=hi)
    T4 = jnp.pad(T4.reshape(Cs * Vc, L, O), ((0, 0), (0, 1), (0, 0)))
    T4 = jnp.pad(T4.reshape(Cs * Vc, 3, 3 * O),
                 ((0, 0), (0, 0), (0, 128 - 3 * O)))
    rows_char = T4.reshape(Cs * Vc, _NCONV)

    w1w = jnp.einsum("ve,hwe->wvh", word_emb, W1f[:, :Dw].reshape(H, Wn, E),
                     precision=hi).reshape(Wn * Vw, H)

    Wchar = (jnp.zeros((256, _NCONV), jnp.float32)
             .at[0:Cs * Vc, :].set(rows_char).astype(jnp.bfloat16))
    Wword = (jnp.zeros((256, 128), jnp.float32)
             .at[0:Wn * Vw, :H].set(w1w).astype(jnp.bfloat16))
    W1c = (jnp.zeros((O, 128), jnp.float32)
           .at[:, :H].set(W1f[:, Dw:].T).astype(jnp.bfloat16))
    W2p = (jnp.zeros((128, 128), jnp.float32)
           .at[:H, :Od].set(W2f.T).astype(jnp.bfloat16))
    aux = (jnp.zeros((8, 128), jnp.float32)
           .at[0, :O].set(conv_b.astype(jnp.float32))
           .at[1, :H].set(b1.astype(jnp.float32))
           .at[2, :].set(-1e30)
           .at[2, :Od].set(b2.astype(jnp.float32)))

    TB = min(tile_b, B)
    grid_b = pl.cdiv(B, TB)

    out = pl.pallas_call(
        functools.partial(_tagger_kernel, O=O, Od=Od),
        out_shape=jax.ShapeDtypeStruct((B, Od), jnp.float32),
        grid_spec=pltpu.PrefetchScalarGridSpec(
            num_scalar_prefetch=0,
            grid=(grid_b,),
            in_specs=[
                pl.BlockSpec((TB, Cs), lambda b: (b, 0)),
                pl.BlockSpec((TB, Wn), lambda b: (b, 0)),
                pl.BlockSpec((Cs, 256), lambda b: (0, 0)),
                pl.BlockSpec((1, 256), lambda b: (0, 0)),
                pl.BlockSpec((Wn, 256), lambda b: (0, 0)),
                pl.BlockSpec((1, 256), lambda b: (0, 0)),
                pl.BlockSpec((256, _NCONV), lambda b: (0, 0)),
                pl.BlockSpec((256, 128), lambda b: (0, 0)),
                pl.BlockSpec((O, 128), lambda b: (0, 0)),
                pl.BlockSpec((128, 128), lambda b: (0, 0)),
                pl.BlockSpec((8, 128), lambda b: (0, 0)),
            ],
            out_specs=pl.BlockSpec((TB, Od), lambda b: (b, 0)),
        ),
        compiler_params=pltpu.CompilerParams(
            dimension_semantics=("parallel",)),
        interpret=interpret,
    )(chars_idxs.astype(jnp.int32), words_idxs.astype(jnp.int32),
      S_c, T_c, S_w, T_w, Wchar, Wword, W1c, W2p, aux)
    return out


def kernel(words_idxs, chars_idxs, char_emb, word_emb, conv_w, conv_b,
           W1, b1, W2, b2):
    return _forward(words_idxs, chars_idxs, char_emb, word_emb,
                    conv_w, conv_b, W1, b1, W2, b2, tile_b=8192)


# speedup vs baseline: 7.8893x; 1.1767x over previous
import functools

import numpy as np

import jax
import jax.numpy as jnp
from jax.experimental import pallas as pl
from jax.experimental.pallas import tpu as pltpu

_NREP = 512
_NCONV = 384
_NBIG = 512


def _tagger_kernel(idx_ref, s_ref, t_ref, wchar_ref, wword_ref,
                   w1c_ref, w2_ref, aux_ref, out_ref, *, O, Od):
    rep = jnp.dot(idx_ref[...], s_ref[...],
                  preferred_element_type=jnp.float32).astype(jnp.bfloat16)
    oh = jnp.where(rep == t_ref[...], jnp.bfloat16(1.0), jnp.bfloat16(0.0))

    conv = jnp.dot(oh[:, 0:256], wchar_ref[...],
                   preferred_element_type=jnp.float32).astype(jnp.bfloat16)
    wordt = jnp.dot(oh[:, 256:512], wword_ref[...],
                    preferred_element_type=jnp.float32)

    g0 = conv[:, 0:128]
    g1 = conv[:, 128:256]
    g2 = conv[:, 256:384]
    gm01 = jnp.maximum(g0, g1)
    gm = jnp.maximum(gm01, g2)
    cf = jnp.maximum(jnp.maximum(gm[:, 0:O], gm[:, O:2 * O]),
                     gm01[:, 2 * O:3 * O]) + aux_ref[0:1, 0:O].astype(jnp.bfloat16)

    h = jnp.tanh(wordt
                 + jnp.dot(cf, w1c_ref[...],
                           preferred_element_type=jnp.float32)
                 + aux_ref[1:2, :])

    logits = jnp.dot(h.astype(jnp.bfloat16), w2_ref[...],
                     preferred_element_type=jnp.float32) + aux_ref[2:3, :]
    lse = jnp.log(jnp.sum(jnp.exp(logits), axis=-1, keepdims=True))
    out_ref[...] = (logits - lse)[:, :Od]


@functools.partial(jax.jit, static_argnames=("tile_b", "interpret"))
def _forward(words_idxs, chars_idxs, char_emb, word_emb, conv_w, conv_b,
             W1, b1, W2, b2, *, tile_b=512, interpret=False):
    B, Cs = chars_idxs.shape
    Wn = words_idxs.shape[1]
    char_emb = char_emb.astype(jnp.float32)
    word_emb = word_emb.astype(jnp.float32)
    Vc, L = char_emb.shape
    Vw, E = word_emb.shape
    Wc = conv_w.astype(jnp.float32)
    O = Wc.shape[0]
    W1f = W1.astype(jnp.float32)
    W2f = W2.astype(jnp.float32)
    Od, H = W2f.shape
    Dw = Wn * E
    hi = jax.lax.Precision.HIGHEST

    S_np = np.zeros((16, 512), np.float32)
    T_np = np.full((1, 512), -1.0, np.float32)
    for c in range(Cs):
        S_np[c, c * Vc:(c + 1) * Vc] = 1.0
        T_np[0, c * Vc:(c + 1) * Vc] = np.arange(Vc)
    for w in range(Wn):
        S_np[Cs + w, 256 + w * Vw:256 + (w + 1) * Vw] = 1.0
        T_np[0, 256 + w * Vw:256 + (w + 1) * Vw] = np.arange(Vw)
    S_c = jnp.asarray(S_np, jnp.bfloat16)
    T_c = jnp.asarray(T_np, jnp.bfloat16)

    Es = jnp.stack([
        jnp.pad(char_emb[:, :L - 1], ((0, 0), (1, 0))),
        char_emb,
        jnp.pad(char_emb[:, 1:], ((0, 0), (0, 1))),
    ], axis=1)
    W3 = jnp.transpose(Wc, (2, 1, 0))
    T4 = jnp.einsum("vkm,kco->cvmo", Es, W3, precision=hi)
    T4 = jnp.pad(T4.reshape(Cs * Vc, L, O), ((0, 0), (0, 1), (0, 0)))
    T4 = jnp.pad(T4.reshape(Cs * Vc, 3, 3 * O),
                 ((0, 0), (0, 0), (0, 128 - 3 * O)))
    rows_char = T4.reshape(Cs * Vc, _NCONV)

    w1w = jnp.einsum("ve,hwe->wvh", word_emb, W1f[:, :Dw].reshape(H, Wn, E),
                     precision=hi).reshape(Wn * Vw, H)

    Wchar = (jnp.zeros((256, _NCONV), jnp.float32)
             .at[0:Cs * Vc, :].set(rows_char).astype(jnp.bfloat16))
    Wword = (jnp.zeros((256, 128), jnp.float32)
             .at[0:Wn * Vw, :H].set(w1w).astype(jnp.bfloat16))
    W1c = (jnp.zeros((O, 128), jnp.float32)
           .at[:, :H].set(W1f[:, Dw:].T).astype(jnp.bfloat16))
    W2p = (jnp.zeros((128, 128), jnp.float32)
           .at[:H, :Od].set(W2f.T).astype(jnp.bfloat16))
    aux = (jnp.zeros((8, 128), jnp.float32)
           .at[0, :O].set(conv_b.astype(jnp.float32))
           .at[1, :H].set(b1.astype(jnp.float32))
           .at[2, :].set(-1e30)
           .at[2, :Od].set(b2.astype(jnp.float32)))

    TB = min(tile_b, B)
    grid_b = pl.cdiv(B, TB)

    idx_all = jnp.pad(
        jnp.concatenate([chars_idxs.astype(jnp.int32),
                         words_idxs.astype(jnp.int32)], axis=1),
        ((0, 0), (0, 16 - Cs - Wn))).astype(jnp.bfloat16)

    out = pl.pallas_call(
        functools.partial(_tagger_kernel, O=O, Od=Od),
        out_shape=jax.ShapeDtypeStruct((B, Od), jnp.float32),
        grid_spec=pltpu.PrefetchScalarGridSpec(
            num_scalar_prefetch=0,
            grid=(grid_b,),
            in_specs=[
                pl.BlockSpec((TB, 16), lambda b: (b, 0)),
                pl.BlockSpec((16, 512), lambda b: (0, 0)),
                pl.BlockSpec((1, 512), lambda b: (0, 0)),
                pl.BlockSpec((256, _NCONV), lambda b: (0, 0)),
                pl.BlockSpec((256, 128), lambda b: (0, 0)),
                pl.BlockSpec((O, 128), lambda b: (0, 0)),
                pl.BlockSpec((128, 128), lambda b: (0, 0)),
                pl.BlockSpec((8, 128), lambda b: (0, 0)),
            ],
            out_specs=pl.BlockSpec((TB, Od), lambda b: (b, 0)),
        ),
        compiler_params=pltpu.CompilerParams(
            dimension_semantics=("parallel",)),
        interpret=interpret,
    )(idx_all, S_c, T_c, Wchar, Wword, W1c, W2p, aux)
    return out


def kernel(words_idxs, chars_idxs, char_emb, word_emb, conv_w, conv_b,
           W1, b1, W2, b2):
    return _forward(words_idxs, chars_idxs, char_emb, word_emb,
                    conv_w, conv_b, W1, b1, W2, b2, tile_b=8192)
